# Initial kernel scaffold; baseline (speedup 1.0000x reference)
import functools

import jax
import jax.numpy as jnp
from jax import lax
from jax.experimental import pallas as pl
from jax.experimental.pallas import tpu as pltpu
from jax.experimental.pallas import tpu_sc as plsc

_D = 96
_SEG = 128
_CHUNK = 1024


def _mlp_table_body(emb_ref, w1_ref, b1_ref, w2_ref, b2_ref, w4_ref, b4_ref,
                    out_ref):
    h = jnp.dot(emb_ref[...], w1_ref[...],
                preferred_element_type=jnp.float32) + b1_ref[...]
    h = jnp.maximum(h, 0.0)
    h = jnp.dot(h, w2_ref[...], preferred_element_type=jnp.float32) + b2_ref[...]
    h = jnp.maximum(h, 0.0)
    out_ref[...] = (jnp.dot(h, w4_ref[...], preferred_element_type=jnp.float32)
                    + b4_ref[...])


def _mlp_table(emb_p, W1, b1, W2, b2, W4, b4):
    return pl.pallas_call(
        _mlp_table_body,
        out_shape=jax.ShapeDtypeStruct((16, _D), jnp.float32),
    )(emb_p, W1, b1.reshape(1, _D), W2, b2.reshape(1, _D),
      W4, b4.reshape(1, _D))


@functools.partial(jax.jit, static_argnums=(2,))
def _sc_gather(table, idx2d, total_rows):
    info = plsc.get_sparse_core_info()
    nw = info.num_cores * info.num_subcores
    nc = info.num_cores
    assert total_rows % (nw * _CHUNK) == 0
    b_per_w = total_rows // nw
    n_chunks = b_per_w // _CHUNK
    segs_per_chunk = _CHUNK // _SEG

    mesh = plsc.VectorSubcoreMesh(core_axis_name="c", subcore_axis_name="s")

    @functools.partial(
        pl.kernel,
        mesh=mesh,
        out_type=jax.ShapeDtypeStruct((total_rows, _D), jnp.float32),
        scratch_types=[
            pltpu.VMEM((segs_per_chunk, _SEG), jnp.int32),
            pltpu.VMEM((_CHUNK, _D), jnp.float32),
            pltpu.SemaphoreType.DMA,
        ],
    )
    def k(table_hbm, idx_hbm, out_hbm, idx_v, rows_v, sem):
        wid = lax.axis_index("s") * nc + lax.axis_index("c")
        base = wid * b_per_w

        def chunk_body(kk, carry):
            off = base + kk * _CHUNK
            row0 = off // _SEG
            pltpu.sync_copy(idx_hbm.at[pl.ds(row0, segs_per_chunk)], idx_v)
            copies = []
            for j in range(segs_per_chunk):
                copies.append(pltpu.async_copy(
                    table_hbm.at[idx_v.at[j]],
                    rows_v.at[pl.ds(j * _SEG, _SEG)],
                    sem))
            for cp in copies:
                cp.wait()
            pltpu.sync_copy(rows_v, out_hbm.at[pl.ds(off, _CHUNK)])
            return carry

        lax.fori_loop(0, n_chunks, chunk_body, 0)

    return k(table, idx2d)


def kernel(x, r, c, emb_input, emb_row, emb_col, W1, b1, W2, b2, W4, b4):
    del r, c, emb_row, emb_col
    n, s = x.shape
    total = n * s
    emb_p = jnp.zeros((16, 16), jnp.float32).at[:emb_input.shape[0]].set(
        emb_input)
    table = _mlp_table(emb_p, W1, b1, W2, b2, W4, b4)
    idx2d = x.astype(jnp.int32).reshape(total // _SEG, _SEG)
    out = _sc_gather(table, idx2d, total)
    return out.reshape(n, s, _D)


# speedup vs baseline: 2.2802x; 2.2802x over previous
import functools

import jax
import jax.numpy as jnp
from jax import lax
from jax.experimental import pallas as pl
from jax.experimental.pallas import tpu as pltpu
from jax.experimental.pallas import tpu_sc as plsc

_D = 96
_SEG = 128
_CHUNK = 1024


def _mlp_table_body(emb_ref, w1_ref, b1_ref, w2_ref, b2_ref, w4_ref, b4_ref,
                    out_ref):
    h = jnp.dot(emb_ref[...], w1_ref[...],
                preferred_element_type=jnp.float32) + b1_ref[...]
    h = jnp.maximum(h, 0.0)
    h = jnp.dot(h, w2_ref[...], preferred_element_type=jnp.float32) + b2_ref[...]
    h = jnp.maximum(h, 0.0)
    out_ref[...] = (jnp.dot(h, w4_ref[...], preferred_element_type=jnp.float32)
                    + b4_ref[...])


def _mlp_table(emb_p, W1, b1, W2, b2, W4, b4):
    return pl.pallas_call(
        _mlp_table_body,
        out_shape=jax.ShapeDtypeStruct((16, _D), jnp.float32),
    )(emb_p, W1, b1.reshape(1, _D), W2, b2.reshape(1, _D),
      W4, b4.reshape(1, _D))


@functools.partial(jax.jit, static_argnums=(2,))
def _sc_gather(table_p, idx_flat, total_rows):
    info = plsc.get_sparse_core_info()
    nw = info.num_cores * info.num_subcores
    nc = info.num_cores
    L = info.num_lanes
    assert total_rows % (nw * _CHUNK) == 0
    b_per_w = total_rows // nw
    n_chunks = b_per_w // _CHUNK
    groups = _CHUNK // L
    jd = _D // L

    mesh = plsc.VectorSubcoreMesh(core_axis_name="c", subcore_axis_name="s")

    @functools.partial(
        pl.kernel,
        mesh=mesh,
        compiler_params=pltpu.CompilerParams(needs_layout_passes=False),
        out_type=jax.ShapeDtypeStruct((total_rows * _D,), jnp.float32),
        scratch_types=[
            pltpu.VMEM((16, 128), jnp.float32),
            pltpu.VMEM((_CHUNK,), jnp.int32),
            pltpu.VMEM((_CHUNK * _D,), jnp.float32),
        ],
    )
    def k(table_hbm, idx_hbm, out_hbm, tbl_v, idx_v, rows_v):
        wid = lax.axis_index("s") * nc + lax.axis_index("c")
        base = wid * b_per_w
        pltpu.sync_copy(table_hbm, tbl_v)
        cols = [jax.lax.iota(jnp.int32, L) + j * L for j in range(jd)]

        def chunk_body(kk, carry):
            off = pl.multiple_of(base + kk * _CHUNK, _CHUNK)
            pltpu.sync_copy(idx_hbm.at[pl.ds(off, _CHUNK)], idx_v)

            def group_body(g, carry2):
                for l in range(L):
                    pos = jnp.full((L,), g * L + l, jnp.int32)
                    rowv = plsc.load_gather(idx_v, [pos])
                    dst = pl.multiple_of(g * (L * _D) + l * _D, L)
                    for j in range(jd):
                        v = plsc.load_gather(tbl_v, [rowv, cols[j]])
                        rows_v[pl.ds(dst + j * L, L)] = v
                return carry2

            lax.fori_loop(0, groups, group_body, 0)
            pltpu.sync_copy(rows_v, out_hbm.at[pl.ds(off * _D, _CHUNK * _D)])
            return carry

        lax.fori_loop(0, n_chunks, chunk_body, 0)

    return k(table_p, idx_flat)


def kernel(x, r, c, emb_input, emb_row, emb_col, W1, b1, W2, b2, W4, b4):
    del r, c, emb_row, emb_col
    n, s = x.shape
    total = n * s
    emb_p = jnp.zeros((16, 16), jnp.float32).at[:emb_input.shape[0]].set(
        emb_input)
    table = _mlp_table(emb_p, W1, b1, W2, b2, W4, b4)
    table_p = jnp.zeros((16, 128), jnp.float32).at[:, :_D].set(table)
    idx_flat = x.astype(jnp.int32).reshape(total)
    out = _sc_gather(table_p, idx_flat, total)
    return out.reshape(n, s, _D)
